# baseline (speedup 1.0000x reference)
import jax
import jax.numpy as jnp
from jax.experimental import pallas as pl
from jax.experimental.pallas import tpu as pltpu

EPS_ = 1e-05
G_ = 64
D_ = 128
BR_ = 10000
CH_ = 512
NCH_ = -(-BR_ // CH_)
TAIL_ = BR_ - (NCH_ - 1) * CH_


def _chunk_onehot(ids_row):
    seg = jax.lax.broadcasted_iota(jnp.int32, (G_, CH_), 0)
    return (seg == ids_row).astype(jnp.bfloat16)


def _bdot(lhs, rhs, dims):
    return jax.lax.dot_general(lhs, rhs, (dims, ((), ())),
                               preferred_element_type=jnp.float32)


def _fused_body(ids_ref, x_ref, ms_ref, w_ref, b_ref, o_ref,
                acc_ref, cnt_ref, ph_ref, plo_ref):
    p = pl.program_id(0)
    i = pl.program_id(1)

    @pl.when((p == 0) & (i == 0))
    def _init():
        acc_ref[...] = jnp.zeros_like(acc_ref)
        cnt_ref[...] = jnp.zeros_like(cnt_ref)

    @pl.when(p == 0)
    def _stats():
        for c in range(NCH_):
            base = min(c * CH_, BR_ - CH_)
            ids_row = ids_ref[0, c, :].reshape(1, CH_)
            oh = _chunk_onehot(ids_row)
            x = x_ref[pl.ds(base, CH_), :]
            x_hi = x.astype(jnp.bfloat16)
            x_lo = (x - x_hi.astype(jnp.float32)).astype(jnp.bfloat16)
            acc_ref[:, :D_] += (
                _bdot(oh, x_hi, ((1,), (0,)))
                + _bdot(oh, x_lo, ((1,), (0,))))
            acc_ref[:, D_:] += _bdot(oh, x_hi * x_hi, ((1,), (0,)))
            cnt = oh[:, :D_]
            for t in range(1, CH_ // D_):
                cnt = cnt + oh[:, t * D_:(t + 1) * D_]
            cnt_ref[...] += cnt.astype(jnp.float32)

    @pl.when((p == 1) & (i == 0))
    def _finalize():
        counts = jnp.maximum(jnp.sum(cnt_ref[...], axis=1), 1.0)
        mean = acc_ref[:, :D_] / counts[:, None]
        m2 = acc_ref[:, D_:] / counts[:, None]
        ms = ms_ref[...]
        a = mean * ms
        var = jnp.maximum(m2 - 2.0 * a * mean + a * a, 0.0)
        s = w_ref[...] * jax.lax.rsqrt(var + EPS_)
        params = jnp.concatenate([s, a * s], axis=1)
        p_hi = params.astype(jnp.bfloat16)
        ph_ref[...] = p_hi
        plo_ref[...] = (params - p_hi.astype(jnp.float32)).astype(jnp.bfloat16)

    @pl.when(p == 1)
    def _apply():
        b = b_ref[...]
        for c in range(NCH_):
            base = min(c * CH_, BR_ - CH_)
            ids_row = ids_ref[0, c, :].reshape(1, CH_)
            oh = _chunk_onehot(ids_row)
            x = x_ref[pl.ds(base, CH_), :]
            g = (_bdot(oh, ph_ref[...], ((0,), (0,)))
                 + _bdot(oh, plo_ref[...], ((0,), (0,))))
            y = x * g[:, :D_] - g[:, D_:] + b
            if c < NCH_ - 1:
                o_ref[pl.ds(base, CH_), :] = y
            else:
                skip = CH_ - TAIL_
                o_ref[pl.ds(base + skip, TAIL_), :] = y[skip:, :]


def kernel(features, weight, bias, mean_scale, segment_ids, num_segments):
    n, d = features.shape
    assert d == D_ and n % BR_ == 0
    nb = n // BR_
    ids = segment_ids.astype(jnp.int32).reshape(nb, BR_)
    head = ids[:, :(NCH_ - 1) * CH_].reshape(nb, NCH_ - 1, CH_)
    tail = jnp.concatenate(
        [jnp.full((nb, 1, CH_ - TAIL_), -1, jnp.int32),
         ids[:, BR_ - TAIL_:].reshape(nb, 1, TAIL_)], axis=2)
    ids_chunks = jnp.concatenate([head, tail], axis=1)
    ms = mean_scale.reshape(1, D_)
    w = weight.reshape(1, D_)
    b = bias.reshape(1, D_)

    out = pl.pallas_call(
        _fused_body,
        grid=(2, nb),
        in_specs=[
            pl.BlockSpec((1, NCH_, CH_), lambda p, i: (i, 0, 0)),
            pl.BlockSpec((BR_, D_), lambda p, i: (i, 0)),
            pl.BlockSpec((1, D_), lambda p, i: (0, 0)),
            pl.BlockSpec((1, D_), lambda p, i: (0, 0)),
            pl.BlockSpec((1, D_), lambda p, i: (0, 0)),
        ],
        out_specs=pl.BlockSpec((BR_, D_), lambda p, i: (i * p, 0)),
        out_shape=jax.ShapeDtypeStruct((n, D_), jnp.float32),
        scratch_shapes=[
            pltpu.VMEM((G_, 2 * D_), jnp.float32),
            pltpu.VMEM((G_, D_), jnp.float32),
            pltpu.VMEM((G_, 2 * D_), jnp.bfloat16),
            pltpu.VMEM((G_, 2 * D_), jnp.bfloat16),
        ],
        compiler_params=pltpu.CompilerParams(
            dimension_semantics=("arbitrary", "arbitrary")),
    )(ids_chunks, features, ms, w, b)
    return out


# speedup vs baseline: 1.1821x; 1.1821x over previous
import jax
import jax.numpy as jnp
from jax.experimental import pallas as pl
from jax.experimental.pallas import tpu as pltpu

EPS_ = 1e-05
G_ = 64
D_ = 128
BR_ = 20000
CH_ = 512
NCH_ = -(-BR_ // CH_)
TAIL_ = BR_ - (NCH_ - 1) * CH_


def _chunk_onehot(ids_row):
    seg = jax.lax.broadcasted_iota(jnp.int32, (G_, CH_), 0)
    return (seg == ids_row).astype(jnp.float32)


def _fused_body(ids_ref, x_ref, ms_ref, w_ref, b_ref, o_ref,
                acc_ref, cnt_ref, params_ref):
    p = pl.program_id(0)
    i = pl.program_id(1)

    @pl.when((p == 0) & (i == 0))
    def _init():
        acc_ref[...] = jnp.zeros_like(acc_ref)
        cnt_ref[...] = jnp.zeros_like(cnt_ref)

    @pl.when(p == 0)
    def _stats():
        for c in range(NCH_):
            base = min(c * CH_, BR_ - CH_)
            ids_row = ids_ref[0, c, :].reshape(1, CH_)
            oh = _chunk_onehot(ids_row)
            x = x_ref[pl.ds(base, CH_), :]
            acc_ref[:, :D_] += jax.lax.dot_general(
                oh, x, (((1,), (0,)), ((), ())),
                preferred_element_type=jnp.float32)
            acc_ref[:, D_:] += jax.lax.dot_general(
                oh, x * x, (((1,), (0,)), ((), ())),
                preferred_element_type=jnp.float32)
            cnt = oh[:, :D_]
            for t in range(1, CH_ // D_):
                cnt = cnt + oh[:, t * D_:(t + 1) * D_]
            cnt_ref[...] += cnt

    @pl.when((p == 1) & (i == 0))
    def _finalize():
        counts = jnp.maximum(jnp.sum(cnt_ref[...], axis=1), 1.0)
        mean = acc_ref[:, :D_] / counts[:, None]
        m2 = acc_ref[:, D_:] / counts[:, None]
        ms = ms_ref[...]
        a = mean * ms
        var = m2 - 2.0 * a * mean + a * a
        s = w_ref[...] * jax.lax.rsqrt(var + EPS_)
        params_ref[:, :D_] = s
        params_ref[:, D_:] = a * s

    @pl.when(p == 1)
    def _apply():
        b = b_ref[...]
        for c in range(NCH_):
            base = min(c * CH_, BR_ - CH_)
            ids_row = ids_ref[0, c, :].reshape(1, CH_)
            oh = _chunk_onehot(ids_row)
            x = x_ref[pl.ds(base, CH_), :]
            g = jax.lax.dot_general(
                oh, params_ref[...], (((0,), (0,)), ((), ())),
                preferred_element_type=jnp.float32)
            y = x * g[:, :D_] - g[:, D_:] + b
            if c < NCH_ - 1:
                o_ref[pl.ds(base, CH_), :] = y
            else:
                skip = CH_ - TAIL_
                o_ref[pl.ds(base + skip, TAIL_), :] = y[skip:, :]


def kernel(features, weight, bias, mean_scale, segment_ids, num_segments):
    n, d = features.shape
    assert d == D_ and n % BR_ == 0
    nb = n // BR_
    ids = segment_ids.astype(jnp.int32).reshape(nb, BR_)
    head = ids[:, :(NCH_ - 1) * CH_].reshape(nb, NCH_ - 1, CH_)
    tail = jnp.concatenate(
        [jnp.full((nb, 1, CH_ - TAIL_), -1, jnp.int32),
         ids[:, BR_ - TAIL_:].reshape(nb, 1, TAIL_)], axis=2)
    ids_chunks = jnp.concatenate([head, tail], axis=1)
    ms = mean_scale.reshape(1, D_)
    w = weight.reshape(1, D_)
    b = bias.reshape(1, D_)

    out = pl.pallas_call(
        _fused_body,
        grid=(2, nb),
        in_specs=[
            pl.BlockSpec((1, NCH_, CH_), lambda p, i: (i, 0, 0)),
            pl.BlockSpec((BR_, D_), lambda p, i: (i, 0)),
            pl.BlockSpec((1, D_), lambda p, i: (0, 0)),
            pl.BlockSpec((1, D_), lambda p, i: (0, 0)),
            pl.BlockSpec((1, D_), lambda p, i: (0, 0)),
        ],
        out_specs=pl.BlockSpec((BR_, D_), lambda p, i: (i * p, 0)),
        out_shape=jax.ShapeDtypeStruct((n, D_), jnp.float32),
        scratch_shapes=[
            pltpu.VMEM((G_, 2 * D_), jnp.float32),
            pltpu.VMEM((G_, D_), jnp.float32),
            pltpu.VMEM((G_, 2 * D_), jnp.float32),
        ],
        compiler_params=pltpu.CompilerParams(
            dimension_semantics=("arbitrary", "arbitrary")),
    )(ids_chunks, features, ms, w, b)
    return out
